# baseline (speedup 1.0000x reference)
import functools

import jax
import jax.numpy as jnp
from jax import lax
from jax.experimental import pallas as pl
from jax.experimental.pallas import tpu as pltpu
from jax.experimental.pallas import tpu_sc as plsc

B, T, D, F = 16, 512, 256, 256
M = 2048
NW = 32
B_PER_W = (B * M) // NW
CH = 128
NCHUNK = B_PER_W // CH
NB = 3
L = 16


BPS = 8
BPS_DP = 4


def _idx_body(t_ref, gidx_ref, tot_ref):
    tt = lax.broadcasted_iota(jnp.int32, (T, T), 0)
    uu = lax.broadcasted_iota(jnp.int32, (T, T), 1)
    tri = (uu <= tt).astype(jnp.float32)
    e_row = lax.broadcasted_iota(jnp.int32, (1, M), 1)
    el = e_row & (CH - 1)
    m_row = (e_row & ~(CH - 1)) + ((el & 15) << 3) + (el >> 4)
    m_f = m_row.astype(jnp.float32)
    ones_row = jnp.full((1, T), 1.0, jnp.float32)
    for i in range(BPS):
        b = pl.program_id(0) * BPS + i
        dur = t_ref[i].astype(jnp.float32)
        cum = lax.dot_general(tri, dur, (((1,), (1,)), ((), ())),
                              preferred_element_type=jnp.float32)
        cmp_f = jnp.where(cum <= m_f, 1.0, 0.0)
        cnt = jnp.dot(ones_row, cmp_f,
                      preferred_element_type=jnp.float32).astype(jnp.int32)
        gidx_ref[i] = jnp.where(cnt < T, b * T + cnt,
                                b * T + (m_row & (T - 1)))
        tot_ref[i] = jnp.broadcast_to(jnp.max(cum.astype(jnp.int32)),
                                      (1, 128))


def _compute_gidx(target):
    t3 = target.reshape(B, 1, T)
    return pl.pallas_call(
        _idx_body,
        grid=(B // BPS,),
        in_specs=[pl.BlockSpec((BPS, 1, T), lambda b: (b, 0, 0))],
        out_specs=[pl.BlockSpec((BPS, 1, M), lambda b: (b, 0, 0)),
                   pl.BlockSpec((BPS, 1, 128), lambda b: (b, 0, 0))],
        out_shape=[jax.ShapeDtypeStruct((B, 1, M), jnp.int32),
                   jax.ShapeDtypeStruct((B, 1, 128), jnp.int32)],
    )(t3)


def _gather(table, gidx3, tot3):
    mesh = plsc.VectorSubcoreMesh(core_axis_name="c", subcore_axis_name="s")

    @functools.partial(
        pl.kernel,
        mesh=mesh,
        out_type=jax.ShapeDtypeStruct((B * M, D), jnp.float32),
        scratch_types=[pltpu.VMEM((CH,), jnp.int32)] * NCHUNK + [
            pltpu.VMEM((128,), jnp.int32),
            pltpu.VMEM((NB, CH, D), jnp.float32),
        ] + [pltpu.SemaphoreType.DMA] * (2 * NB),
    )
    def k(table_hbm, idx_hbm, tot_hbm, out_hbm, *refs):
        idx_refs = refs[:NCHUNK]
        tot_v = refs[NCHUNK]
        bufs = refs[NCHUNK + 1]
        sems = refs[NCHUNK + 2:]
        gsems, osems = sems[:NB], sems[NB:]
        wid = lax.axis_index("c") * 16 + lax.axis_index("s")
        b = wid // 2
        m0 = (wid % 2) * B_PER_W
        pltpu.sync_copy(tot_hbm.at[b, 0], tot_v)
        for c in range(NCHUNK):
            pltpu.sync_copy(idx_hbm.at[b, 0, pl.ds(m0 + c * CH, CH)],
                            idx_refs[c])
        total = tot_v[pl.ds(0, L)][0]
        vlim = jnp.clip(total - m0, 0, B_PER_W)

        def gather(c):
            cond = c * CH < vlim

            @pl.when(cond)
            def _():
                pltpu.async_copy(
                    table_hbm.at[idx_refs[c]], bufs.at[c % NB], gsems[c % NB])

            return cond

        def wait_gather(c, cond):
            @pl.when(cond)
            def _():
                pltpu.make_async_copy(
                    table_hbm.at[idx_refs[c]], bufs.at[c % NB],
                    gsems[c % NB]).wait()

        lanes = lax.broadcasted_iota(jnp.int32, (L,), 0)

        def put(c):
            base = wid * B_PER_W + c * CH
            cps = []
            for v in range(CH // L):
                dst = base + lanes * 8 + v
                cps.append(pltpu.async_copy(
                    bufs.at[c % NB, pl.ds(v * L, L)],
                    out_hbm.at[dst], osems[c % NB]))
            return cps

        zrow = jnp.zeros((L,), jnp.float32)

        def zero_tail(c):
            lo = jnp.clip(vlim - c * CH, 0, CH)

            def body(f, _):
                r = (f % 8) * L + f // 8
                for gg in range(D // L):
                    bufs[c % NB, r, pl.ds(gg * L, L)] = zrow
                return 0

            lax.fori_loop(lo, CH, body, 0)

        gcond = [None] * NCHUNK
        ocp = [None] * NCHUNK
        for c in range(NB):
            gcond[c] = gather(c)
        for c in range(NCHUNK):
            wait_gather(c, gcond[c])
            zero_tail(c)
            ocp[c] = put(c)
            if c >= 1 and c + NB - 1 < NCHUNK:
                for cp in ocp[c - 1]:
                    cp.wait()
                gcond[c + NB - 1] = gather(c + NB - 1)
        for c in range(max(0, NCHUNK - NB), NCHUNK):
            for cp in ocp[c]:
                cp.wait()

    return k(table, gidx3, tot3)


def _dp_body(x_ref, w1_ref, b1_ref, g1_ref, be1_ref, w2_ref, b2_ref, g2_ref,
             be2_ref, lw_ref, lb_ref, dp_ref):
    ones_col = jnp.full((F, 1), 1.0 / F, jnp.float32)

    def conv_ln(h, w_ref, b_ref, g_ref, be_ref):
        row = lax.broadcasted_iota(jnp.int32, (T, 1), 0)
        nf = w_ref.shape[0] // 3
        a0 = jnp.dot(h, w_ref[pl.ds(0, nf)],
                     preferred_element_type=jnp.float32)
        a1 = jnp.dot(h, w_ref[pl.ds(nf, nf)],
                     preferred_element_type=jnp.float32)
        a2 = jnp.dot(h, w_ref[pl.ds(2 * nf, nf)],
                     preferred_element_type=jnp.float32)
        y = (jnp.where(row == 0, 0.0, pltpu.roll(a0, 1, 0)) + a1
             + jnp.where(row == T - 1, 0.0, pltpu.roll(a2, T - 1, 0))
             + b_ref[...])
        y = jnp.maximum(y, 0.0)
        mu = jnp.dot(y, ones_col, preferred_element_type=jnp.float32)
        var = jnp.dot(y * y, ones_col,
                      preferred_element_type=jnp.float32) - mu * mu
        return (y - mu) * lax.rsqrt(var + 1e-5) * g_ref[...] + be_ref[...]

    for i in range(BPS_DP):
        h = conv_ln(x_ref[i], w1_ref, b1_ref, g1_ref, be1_ref)
        h = conv_ln(h, w2_ref, b2_ref, g2_ref, be2_ref)
        dp = lax.dot_general(lw_ref[...], h, (((1,), (1,)), ((), ())),
                             preferred_element_type=jnp.float32)
        dp_ref[i] = dp + lb_ref[0, 0]


def _duration_predictor(x, w1c, b1, g1, be1, w2c, b2, g2, be2, lw, lb):
    full = lambda s: pl.BlockSpec(s, lambda b: tuple(0 for _ in s))
    dp3 = pl.pallas_call(
        _dp_body,
        grid=(B // BPS_DP,),
        in_specs=[
            pl.BlockSpec((BPS_DP, T, D), lambda b: (b, 0, 0)),
            full((3 * D, F)), full((1, F)), full((1, F)), full((1, F)),
            full((3 * F, F)), full((1, F)), full((1, F)), full((1, F)),
            full((1, F)), full((1, 1)),
        ],
        out_specs=pl.BlockSpec((BPS_DP, 1, T), lambda b: (b, 0, 0)),
        out_shape=jax.ShapeDtypeStruct((B, 1, T), jnp.float32),
    )(x, w1c, b1, g1, be1, w2c, b2, g2, be2, lw, lb)
    return dp3.reshape(B, T)


def kernel(x, target, mel_max_length, conv1_w, conv1_b, ln1_g, ln1_b, conv2_w,
           conv2_b, ln2_g, ln2_b, lin_w, lin_b):
    gidx, tot = _compute_gidx(target)
    eps = (tot[0, 0, 0] * 0).astype(x.dtype)
    w1c = conv1_w.transpose(2, 1, 0).reshape(3 * D, F) + eps
    w2c = conv2_w.transpose(2, 1, 0).reshape(3 * F, F)
    b1 = conv1_b.reshape(1, F)
    b2 = conv2_b.reshape(1, F)
    g1, be1 = ln1_g.reshape(1, F), ln1_b.reshape(1, F)
    g2, be2 = ln2_g.reshape(1, F), ln2_b.reshape(1, F)
    lb = lin_b.reshape(1, 1)

    out_flat = _gather(x.reshape(B * T, D), gidx, tot)
    out = out_flat.reshape(B, M, D)
    dp = _duration_predictor(x, w1c, b1, g1, be1, w2c, b2, g2, be2, lin_w, lb)
    return (out, dp)


# speedup vs baseline: 1.0429x; 1.0429x over previous
import functools

import jax
import jax.numpy as jnp
from jax import lax
from jax.experimental import pallas as pl
from jax.experimental.pallas import tpu as pltpu
from jax.experimental.pallas import tpu_sc as plsc

B, T, D, F = 16, 512, 256, 256
M = 2048
NW = 32
B_PER_W = (B * M) // NW
CH = 128
NCHUNK = B_PER_W // CH
NB = 3
L = 16


BPS = 8
BPS_DP = 4


def _idx_body(t_ref, gidx_ref, tot_ref):
    tt = lax.broadcasted_iota(jnp.int32, (T, T), 0)
    uu = lax.broadcasted_iota(jnp.int32, (T, T), 1)
    tri = (uu <= tt).astype(jnp.float32)
    e_row = lax.broadcasted_iota(jnp.int32, (1, M), 1)
    el = e_row & (CH - 1)
    m_row = (e_row & ~(CH - 1)) + ((el & 15) << 3) + (el >> 4)
    m_f = m_row.astype(jnp.float32)
    ones_row = jnp.full((1, T), 1.0, jnp.float32)
    for i in range(BPS):
        b = pl.program_id(0) * BPS + i
        dur = t_ref[i].astype(jnp.float32)
        cum = lax.dot_general(tri, dur, (((1,), (1,)), ((), ())),
                              preferred_element_type=jnp.float32)
        cmp_f = jnp.where(cum <= m_f, 1.0, 0.0)
        cnt = jnp.dot(ones_row, cmp_f,
                      preferred_element_type=jnp.float32).astype(jnp.int32)
        gidx_ref[i] = jnp.where(cnt < T, b * T + cnt,
                                b * T + (m_row & (T - 1)))
        tot_ref[i] = jnp.broadcast_to(jnp.max(cum.astype(jnp.int32)),
                                      (1, 128))


def _compute_gidx(target):
    t3 = target.reshape(B, 1, T)
    return pl.pallas_call(
        _idx_body,
        grid=(B // BPS,),
        in_specs=[pl.BlockSpec((BPS, 1, T), lambda b: (b, 0, 0))],
        out_specs=[pl.BlockSpec((BPS, 1, M), lambda b: (b, 0, 0)),
                   pl.BlockSpec((BPS, 1, 128), lambda b: (b, 0, 0))],
        out_shape=[jax.ShapeDtypeStruct((B, 1, M), jnp.int32),
                   jax.ShapeDtypeStruct((B, 1, 128), jnp.int32)],
    )(t3)


def _gather(table, gidx3, tot3):
    mesh = plsc.VectorSubcoreMesh(core_axis_name="c", subcore_axis_name="s")

    @functools.partial(
        pl.kernel,
        mesh=mesh,
        out_type=jax.ShapeDtypeStruct((B * M, D), jnp.float32),
        scratch_types=[pltpu.VMEM((CH,), jnp.int32)] * NCHUNK + [
            pltpu.VMEM((128,), jnp.int32),
            pltpu.VMEM((NB, CH, D), jnp.float32),
        ] + [pltpu.SemaphoreType.DMA] * (2 * NB + 1),
    )
    def k(table_hbm, idx_hbm, tot_hbm, out_hbm, *refs):
        idx_refs = refs[:NCHUNK]
        tot_v = refs[NCHUNK]
        bufs = refs[NCHUNK + 1]
        sems = refs[NCHUNK + 2:]
        gsems, osems, isem = sems[:NB], sems[NB:2 * NB], sems[2 * NB]
        wid = lax.axis_index("c") * 16 + lax.axis_index("s")
        b = wid // 2
        m0 = (wid % 2) * B_PER_W
        icps = [pltpu.async_copy(tot_hbm.at[b, 0], tot_v, isem)]
        for c in range(NCHUNK):
            icps.append(pltpu.async_copy(
                idx_hbm.at[b, 0, pl.ds(m0 + c * CH, CH)], idx_refs[c], isem))
        for cp in icps:
            cp.wait()
        total = tot_v[pl.ds(0, L)][0]
        vlim = jnp.clip(total - m0, 0, B_PER_W)

        def gather(c):
            cond = c * CH < vlim

            @pl.when(cond)
            def _():
                pltpu.async_copy(
                    table_hbm.at[idx_refs[c]], bufs.at[c % NB], gsems[c % NB])

            return cond

        def wait_gather(c, cond):
            @pl.when(cond)
            def _():
                pltpu.make_async_copy(
                    table_hbm.at[idx_refs[c]], bufs.at[c % NB],
                    gsems[c % NB]).wait()

        lanes = lax.broadcasted_iota(jnp.int32, (L,), 0)

        def put(c):
            base = wid * B_PER_W + c * CH
            cps = []
            for v in range(CH // L):
                dst = base + lanes * 8 + v
                cps.append(pltpu.async_copy(
                    bufs.at[c % NB, pl.ds(v * L, L)],
                    out_hbm.at[dst], osems[c % NB]))
            return cps

        zrow = jnp.zeros((L,), jnp.float32)

        def zero_tail(c):
            lo = jnp.clip(vlim - c * CH, 0, CH)

            def body(f, _):
                r = (f % 8) * L + f // 8
                for gg in range(D // L):
                    bufs[c % NB, r, pl.ds(gg * L, L)] = zrow
                return 0

            lax.fori_loop(lo, CH, body, 0)

        gcond = [None] * NCHUNK
        ocp = [None] * NCHUNK
        for c in range(NB):
            gcond[c] = gather(c)
        for c in range(NCHUNK):
            wait_gather(c, gcond[c])
            zero_tail(c)
            ocp[c] = put(c)
            if c >= 1 and c + NB - 1 < NCHUNK:
                for cp in ocp[c - 1]:
                    cp.wait()
                gcond[c + NB - 1] = gather(c + NB - 1)
        for c in range(max(0, NCHUNK - NB), NCHUNK):
            for cp in ocp[c]:
                cp.wait()

    return k(table, gidx3, tot3)


def _dp_body(x_ref, w1_ref, b1_ref, g1_ref, be1_ref, w2_ref, b2_ref, g2_ref,
             be2_ref, lw_ref, lb_ref, dp_ref):
    ones_col = jnp.full((F, 1), 1.0 / F, jnp.float32)

    def conv_ln(h, w_ref, b_ref, g_ref, be_ref):
        row = lax.broadcasted_iota(jnp.int32, (T, 1), 0)
        nf = w_ref.shape[0] // 3
        a0 = jnp.dot(h, w_ref[pl.ds(0, nf)],
                     preferred_element_type=jnp.float32)
        a1 = jnp.dot(h, w_ref[pl.ds(nf, nf)],
                     preferred_element_type=jnp.float32)
        a2 = jnp.dot(h, w_ref[pl.ds(2 * nf, nf)],
                     preferred_element_type=jnp.float32)
        y = (jnp.where(row == 0, 0.0, pltpu.roll(a0, 1, 0)) + a1
             + jnp.where(row == T - 1, 0.0, pltpu.roll(a2, T - 1, 0))
             + b_ref[...])
        y = jnp.maximum(y, 0.0)
        mu = jnp.dot(y, ones_col, preferred_element_type=jnp.float32)
        var = jnp.dot(y * y, ones_col,
                      preferred_element_type=jnp.float32) - mu * mu
        return (y - mu) * lax.rsqrt(var + 1e-5) * g_ref[...] + be_ref[...]

    for i in range(BPS_DP):
        h = conv_ln(x_ref[i], w1_ref, b1_ref, g1_ref, be1_ref)
        h = conv_ln(h, w2_ref, b2_ref, g2_ref, be2_ref)
        dp = lax.dot_general(lw_ref[...], h, (((1,), (1,)), ((), ())),
                             preferred_element_type=jnp.float32)
        dp_ref[i] = dp + lb_ref[0, 0]


def _duration_predictor(x, w1c, b1, g1, be1, w2c, b2, g2, be2, lw, lb):
    full = lambda s: pl.BlockSpec(s, lambda b: tuple(0 for _ in s))
    dp3 = pl.pallas_call(
        _dp_body,
        grid=(B // BPS_DP,),
        in_specs=[
            pl.BlockSpec((BPS_DP, T, D), lambda b: (b, 0, 0)),
            full((3 * D, F)), full((1, F)), full((1, F)), full((1, F)),
            full((3 * F, F)), full((1, F)), full((1, F)), full((1, F)),
            full((1, F)), full((1, 1)),
        ],
        out_specs=pl.BlockSpec((BPS_DP, 1, T), lambda b: (b, 0, 0)),
        out_shape=jax.ShapeDtypeStruct((B, 1, T), jnp.float32),
    )(x, w1c, b1, g1, be1, w2c, b2, g2, be2, lw, lb)
    return dp3.reshape(B, T)


def kernel(x, target, mel_max_length, conv1_w, conv1_b, ln1_g, ln1_b, conv2_w,
           conv2_b, ln2_g, ln2_b, lin_w, lin_b):
    gidx, tot = _compute_gidx(target)
    eps = (tot[0, 0, 0] * 0).astype(x.dtype)
    w1c = conv1_w.transpose(2, 1, 0).reshape(3 * D, F) + eps
    w2c = conv2_w.transpose(2, 1, 0).reshape(3 * F, F)
    b1 = conv1_b.reshape(1, F)
    b2 = conv2_b.reshape(1, F)
    g1, be1 = ln1_g.reshape(1, F), ln1_b.reshape(1, F)
    g2, be2 = ln2_g.reshape(1, F), ln2_b.reshape(1, F)
    lb = lin_b.reshape(1, 1)

    out_flat = _gather(x.reshape(B * T, D), gidx, tot)
    out = out_flat.reshape(B, M, D)
    dp = _duration_predictor(x, w1c, b1, g1, be1, w2c, b2, g2, be2, lin_w, lb)
    return (out, dp)


# speedup vs baseline: 1.0462x; 1.0031x over previous
import functools

import jax
import jax.numpy as jnp
from jax import lax
from jax.experimental import pallas as pl
from jax.experimental.pallas import tpu as pltpu
from jax.experimental.pallas import tpu_sc as plsc

B, T, D, F = 16, 512, 256, 256
M = 2048
NW = 32
B_PER_W = (B * M) // NW
CH = 128
NCHUNK = B_PER_W // CH
NB = 3
L = 16


BPS = 16
BPS_DP = 4


def _idx_body(t_ref, gidx_ref, tot_ref):
    tt = lax.broadcasted_iota(jnp.int32, (T, T), 0)
    uu = lax.broadcasted_iota(jnp.int32, (T, T), 1)
    tri = (uu <= tt).astype(jnp.float32)
    e_row = lax.broadcasted_iota(jnp.int32, (1, M), 1)
    el = e_row & (CH - 1)
    m_row = (e_row & ~(CH - 1)) + ((el & 15) << 3) + (el >> 4)
    m_f = m_row.astype(jnp.float32)
    ones_row = jnp.full((1, T), 1.0, jnp.float32)
    for i in range(BPS):
        b = pl.program_id(0) * BPS + i
        dur = t_ref[i].astype(jnp.float32)
        cum = lax.dot_general(tri, dur, (((1,), (1,)), ((), ())),
                              preferred_element_type=jnp.float32)
        cmp_f = jnp.where(cum <= m_f, 1.0, 0.0)
        cnt = jnp.dot(ones_row, cmp_f,
                      preferred_element_type=jnp.float32).astype(jnp.int32)
        gidx_ref[i] = jnp.where(cnt < T, b * T + cnt,
                                b * T + (m_row & (T - 1)))
        tot_ref[i] = jnp.broadcast_to(jnp.max(cum.astype(jnp.int32)),
                                      (1, 128))


def _compute_gidx(target):
    t3 = target.reshape(B, 1, T)
    return pl.pallas_call(
        _idx_body,
        grid=(B // BPS,),
        in_specs=[pl.BlockSpec((BPS, 1, T), lambda b: (b, 0, 0))],
        out_specs=[pl.BlockSpec((BPS, 1, M), lambda b: (b, 0, 0)),
                   pl.BlockSpec((BPS, 1, 128), lambda b: (b, 0, 0))],
        out_shape=[jax.ShapeDtypeStruct((B, 1, M), jnp.int32),
                   jax.ShapeDtypeStruct((B, 1, 128), jnp.int32)],
    )(t3)


def _gather(table, gidx3, tot3):
    mesh = plsc.VectorSubcoreMesh(core_axis_name="c", subcore_axis_name="s")

    @functools.partial(
        pl.kernel,
        mesh=mesh,
        out_type=jax.ShapeDtypeStruct((B * M, D), jnp.float32),
        scratch_types=[pltpu.VMEM((CH,), jnp.int32)] * NCHUNK + [
            pltpu.VMEM((128,), jnp.int32),
            pltpu.VMEM((NB, CH, D), jnp.float32),
        ] + [pltpu.SemaphoreType.DMA] * (2 * NB + 1),
    )
    def k(table_hbm, idx_hbm, tot_hbm, out_hbm, *refs):
        idx_refs = refs[:NCHUNK]
        tot_v = refs[NCHUNK]
        bufs = refs[NCHUNK + 1]
        sems = refs[NCHUNK + 2:]
        gsems, osems, isem = sems[:NB], sems[NB:2 * NB], sems[2 * NB]
        wid = lax.axis_index("c") * 16 + lax.axis_index("s")
        b = wid // 2
        m0 = (wid % 2) * B_PER_W
        icps = [pltpu.async_copy(tot_hbm.at[b, 0], tot_v, isem)]
        for c in range(NCHUNK):
            icps.append(pltpu.async_copy(
                idx_hbm.at[b, 0, pl.ds(m0 + c * CH, CH)], idx_refs[c], isem))
        for cp in icps:
            cp.wait()
        total = tot_v[pl.ds(0, L)][0]
        vlim = jnp.clip(total - m0, 0, B_PER_W)

        def gather(c):
            cond = c * CH < vlim

            @pl.when(cond)
            def _():
                pltpu.async_copy(
                    table_hbm.at[idx_refs[c]], bufs.at[c % NB], gsems[c % NB])

            return cond

        def wait_gather(c, cond):
            @pl.when(cond)
            def _():
                pltpu.make_async_copy(
                    table_hbm.at[idx_refs[c]], bufs.at[c % NB],
                    gsems[c % NB]).wait()

        lanes = lax.broadcasted_iota(jnp.int32, (L,), 0)

        def put(c):
            base = wid * B_PER_W + c * CH
            cps = []
            for v in range(CH // L):
                dst = base + lanes * 8 + v
                cps.append(pltpu.async_copy(
                    bufs.at[c % NB, pl.ds(v * L, L)],
                    out_hbm.at[dst], osems[c % NB]))
            return cps

        zrow = jnp.zeros((L,), jnp.float32)

        def zero_tail(c):
            lo = jnp.clip(vlim - c * CH, 0, CH)

            def body(f, _):
                r = (f % 8) * L + f // 8
                for gg in range(D // L):
                    bufs[c % NB, r, pl.ds(gg * L, L)] = zrow
                return 0

            lax.fori_loop(lo, CH, body, 0)

        gcond = [None] * NCHUNK
        ocp = [None] * NCHUNK
        for c in range(NB):
            gcond[c] = gather(c)
        for c in range(NCHUNK):
            wait_gather(c, gcond[c])
            zero_tail(c)
            ocp[c] = put(c)
            if c >= 1 and c + NB - 1 < NCHUNK:
                for cp in ocp[c - 1]:
                    cp.wait()
                gcond[c + NB - 1] = gather(c + NB - 1)
        for c in range(max(0, NCHUNK - NB), NCHUNK):
            for cp in ocp[c]:
                cp.wait()

    return k(table, gidx3, tot3)


def _dp_body(x_ref, w1_ref, b1_ref, g1_ref, be1_ref, w2_ref, b2_ref, g2_ref,
             be2_ref, lw_ref, lb_ref, dp_ref):
    ones_col = jnp.full((F, 1), 1.0 / F, jnp.float32)

    def conv_ln(h, w_ref, b_ref, g_ref, be_ref):
        row = lax.broadcasted_iota(jnp.int32, (T, 1), 0)
        nf = w_ref.shape[0] // 3
        a0 = jnp.dot(h, w_ref[pl.ds(0, nf)],
                     preferred_element_type=jnp.float32)
        a1 = jnp.dot(h, w_ref[pl.ds(nf, nf)],
                     preferred_element_type=jnp.float32)
        a2 = jnp.dot(h, w_ref[pl.ds(2 * nf, nf)],
                     preferred_element_type=jnp.float32)
        y = (jnp.where(row == 0, 0.0, pltpu.roll(a0, 1, 0)) + a1
             + jnp.where(row == T - 1, 0.0, pltpu.roll(a2, T - 1, 0))
             + b_ref[...])
        y = jnp.maximum(y, 0.0)
        mu = jnp.dot(y, ones_col, preferred_element_type=jnp.float32)
        var = jnp.dot(y * y, ones_col,
                      preferred_element_type=jnp.float32) - mu * mu
        return (y - mu) * lax.rsqrt(var + 1e-5) * g_ref[...] + be_ref[...]

    for i in range(BPS_DP):
        h = conv_ln(x_ref[i], w1_ref, b1_ref, g1_ref, be1_ref)
        h = conv_ln(h, w2_ref, b2_ref, g2_ref, be2_ref)
        dp = lax.dot_general(lw_ref[...], h, (((1,), (1,)), ((), ())),
                             preferred_element_type=jnp.float32)
        dp_ref[i] = dp + lb_ref[0, 0]


def _duration_predictor(x, w1c, b1, g1, be1, w2c, b2, g2, be2, lw, lb):
    full = lambda s: pl.BlockSpec(s, lambda b: tuple(0 for _ in s))
    dp3 = pl.pallas_call(
        _dp_body,
        grid=(B // BPS_DP,),
        in_specs=[
            pl.BlockSpec((BPS_DP, T, D), lambda b: (b, 0, 0)),
            full((3 * D, F)), full((1, F)), full((1, F)), full((1, F)),
            full((3 * F, F)), full((1, F)), full((1, F)), full((1, F)),
            full((1, F)), full((1, 1)),
        ],
        out_specs=pl.BlockSpec((BPS_DP, 1, T), lambda b: (b, 0, 0)),
        out_shape=jax.ShapeDtypeStruct((B, 1, T), jnp.float32),
    )(x, w1c, b1, g1, be1, w2c, b2, g2, be2, lw, lb)
    return dp3.reshape(B, T)


def kernel(x, target, mel_max_length, conv1_w, conv1_b, ln1_g, ln1_b, conv2_w,
           conv2_b, ln2_g, ln2_b, lin_w, lin_b):
    gidx, tot = _compute_gidx(target)
    eps = (tot[0, 0, 0] * 0).astype(x.dtype)
    w1c = conv1_w.transpose(2, 1, 0).reshape(3 * D, F) + eps
    w2c = conv2_w.transpose(2, 1, 0).reshape(3 * F, F)
    b1 = conv1_b.reshape(1, F)
    b2 = conv2_b.reshape(1, F)
    g1, be1 = ln1_g.reshape(1, F), ln1_b.reshape(1, F)
    g2, be2 = ln2_g.reshape(1, F), ln2_b.reshape(1, F)
    lb = lin_b.reshape(1, 1)

    out_flat = _gather(x.reshape(B * T, D), gidx, tot)
    out = out_flat.reshape(B, M, D)
    dp = _duration_predictor(x, w1c, b1, g1, be1, w2c, b2, g2, be2, lin_w, lb)
    return (out, dp)
